# baseline (speedup 1.0000x reference)
import jax
import jax.numpy as jnp
from jax import lax
from jax.experimental import pallas as pl
from jax.experimental.pallas import tpu as pltpu
from jax.experimental.pallas import tpu_sc as plsc

N = 500000
F = 32
NBIN = 256
HIST = F * NBIN

NTILES = 32
ROWS_PER_TILE = 15616
CH = 512
NCHUNK = 30
NBUF = 3
NOUTER = 10
HALF_CH = 256
TAIL_BASE = NTILES * ROWS_PER_TILE
TAIL1 = 256
TAIL2 = 32
GBUF = CH + 16
XPAD = CH


def _sc_body(xt_hbm, g_hbm, h_hbm, pg_hbm, ph_hbm,
             xb0, xb1, xb2, gb0, gb1, gb2, hb0, hb1, hb2,
             histg, histh, sem0, sem1, sem2):
  c = lax.axis_index("c")
  s = lax.axis_index("s")
  wid = s * 2 + c
  base = wid * ROWS_PER_TILE

  zeros16 = jnp.zeros((16,), jnp.float32)

  xbufs = (xb0, xb1, xb2)
  gbufs = (gb0, gb1, gb2)
  hbufs = (hb0, hb1, hb2)
  sems = (sem0, sem1, sem2)

  def start(st, nrows, slot):
    pltpu.async_copy(xt_hbm.at[:, pl.ds(st, nrows)],
                     xbufs[slot].at[:, pl.ds(0, nrows)], sems[slot])
    pltpu.async_copy(g_hbm.at[pl.ds(st, nrows)],
                     gbufs[slot].at[pl.ds(0, nrows)], sems[slot])
    pltpu.async_copy(h_hbm.at[pl.ds(st, nrows)],
                     hbufs[slot].at[pl.ds(0, nrows)], sems[slot])

  def wait_slot(slot, nrows):
    pltpu.make_async_copy(xt_hbm.at[:, pl.ds(0, nrows)],
                          xbufs[slot].at[:, pl.ds(0, nrows)],
                          sems[slot]).wait()
    pltpu.make_async_copy(g_hbm.at[pl.ds(0, nrows)],
                          gbufs[slot].at[pl.ds(0, nrows)], sems[slot]).wait()
    pltpu.make_async_copy(h_hbm.at[pl.ds(0, nrows)],
                          hbufs[slot].at[pl.ds(0, nrows)], sems[slot]).wait()

  def process(slot, nrows):
    xb = xbufs[slot]
    gb = gbufs[slot]
    hb = hbufs[slot]

    @plsc.parallel_loop(0, nrows // 16, 1, unroll=2)
    def _grp(gi):
      r = gi * 16
      gv = gb[pl.ds(r, 16)]
      hv = hb[pl.ds(r, 16)]
      for f in range(0, F, 4):
        xvs = [xb[f + j, pl.ds(r, 16)] for j in range(4)]
        idxs = [xvs[j] + ((f + j) * NBIN) for j in range(4)]
        for j in range(4):
          plsc.addupdate_scatter(histg, [idxs[j]], gv)
        for j in range(4):
          plsc.addupdate_scatter(histh, [idxs[j]], hv)

  start(base, CH, 0)
  start(base + CH, CH, 1)

  @plsc.parallel_loop(0, HIST // 16, 1, unroll=8)
  def _zero(i):
    histg[pl.ds(i * 16, 16)] = zeros16
    histh[pl.ds(i * 16, 16)] = zeros16

  def outer(j, carry):
    for k in range(NBUF):
      ci = j * NBUF + k
      wait_slot(k, CH)

      @pl.when(ci + 2 < NCHUNK)
      def _():
        start(base + (ci + 2) * CH, CH, (k + 2) % NBUF)

      @pl.when(ci + 2 == NCHUNK)
      def _():
        start(base + NCHUNK * CH, HALF_CH, (k + 2) % NBUF)

      process(k, CH)
    return carry
  lax.fori_loop(0, NOUTER, outer, 0)

  wait_slot(0, HALF_CH)
  process(0, HALF_CH)

  @pl.when(wid == 0)
  def _tail():
    start(TAIL_BASE, TAIL1, 1)
    wait_slot(1, TAIL1)
    process(1, TAIL1)

  pltpu.sync_copy(histg, pg_hbm.at[pl.ds(wid * HIST, HIST)])
  pltpu.sync_copy(histh, ph_hbm.at[pl.ds(wid * HIST, HIST)])


def _tc_body(pg_ref, ph_ref, xt_ref, gt_ref, ht_ref, gl_ref, hl_ref):
  xt = jnp.broadcast_to(xt_ref[...][:, :, None], (TAIL2, F, NBIN))
  bins = lax.broadcasted_iota(jnp.int32, (TAIL2, F, NBIN), 2)
  m = (xt == bins).astype(jnp.float32)
  hg_t = jnp.sum(m * gt_ref[...][:, None, None], axis=0)
  hh_t = jnp.sum(m * ht_ref[...][:, None, None], axis=0)
  hg = jnp.sum(pg_ref[...], axis=0) + hg_t
  hh = jnp.sum(ph_ref[...], axis=0) + hh_t
  rows = lax.broadcasted_iota(jnp.int32, (NBIN, NBIN), 0)
  cols = lax.broadcasted_iota(jnp.int32, (NBIN, NBIN), 1)
  tri = (rows <= cols).astype(jnp.float32)
  gl_ref[...] = jnp.dot(hg, tri, preferred_element_type=jnp.float32)
  hl_ref[...] = jnp.dot(hh, tri, preferred_element_type=jnp.float32)


@jax.jit
def kernel(X, gradient, hessian):
  mesh = plsc.VectorSubcoreMesh(core_axis_name="c", subcore_axis_name="s")
  sc = pl.kernel(
      _sc_body,
      out_type=(
          jax.ShapeDtypeStruct((NTILES * HIST,), jnp.float32),
          jax.ShapeDtypeStruct((NTILES * HIST,), jnp.float32),
      ),
      mesh=mesh,
      compiler_params=pltpu.CompilerParams(needs_layout_passes=False),
      scratch_types=[
          pltpu.VMEM((F, XPAD), jnp.int32),
          pltpu.VMEM((F, XPAD), jnp.int32),
          pltpu.VMEM((F, XPAD), jnp.int32),
          pltpu.VMEM((GBUF,), jnp.float32),
          pltpu.VMEM((GBUF,), jnp.float32),
          pltpu.VMEM((GBUF,), jnp.float32),
          pltpu.VMEM((GBUF,), jnp.float32),
          pltpu.VMEM((GBUF,), jnp.float32),
          pltpu.VMEM((GBUF,), jnp.float32),
          pltpu.VMEM((HIST,), jnp.float32),
          pltpu.VMEM((HIST,), jnp.float32),
          pltpu.SemaphoreType.DMA,
          pltpu.SemaphoreType.DMA,
          pltpu.SemaphoreType.DMA,
      ],
  )
  pg, ph = sc(X.T, gradient, hessian)

  pg3 = pg.reshape(NTILES, F, NBIN)
  ph3 = ph.reshape(NTILES, F, NBIN)
  gl, hl = pl.pallas_call(
      _tc_body,
      out_shape=(
          jax.ShapeDtypeStruct((F, NBIN), jnp.float32),
          jax.ShapeDtypeStruct((F, NBIN), jnp.float32),
      ),
  )(pg3, ph3, X[TAIL_BASE + TAIL1:], gradient[TAIL_BASE + TAIL1:],
    hessian[TAIL_BASE + TAIL1:])
  return (gl[None], hl[None])


# speedup vs baseline: 1.0191x; 1.0191x over previous
import jax
import jax.numpy as jnp
from jax import lax
from jax.experimental import pallas as pl
from jax.experimental.pallas import tpu as pltpu
from jax.experimental.pallas import tpu_sc as plsc

N = 500000
F = 32
NBIN = 256
HIST = F * NBIN

NTILES = 32
ROWS_PER_TILE = 15616
CH = 512
NCHUNK = 30
NBUF = 3
NOUTER = 10
HALF_CH = 256
TAIL_BASE = NTILES * ROWS_PER_TILE
TAIL1 = 256
TAIL2 = 32
GBUF = CH + 16
XPAD = CH


def _sc_body(xt_hbm, g_hbm, h_hbm, pg_hbm, ph_hbm,
             xb0, xb1, xb2, gb0, gb1, gb2, hb0, hb1, hb2,
             histg, histh, sem0, sem1, sem2):
  c = lax.axis_index("c")
  s = lax.axis_index("s")
  wid = s * 2 + c
  base = wid * ROWS_PER_TILE

  zeros16 = jnp.zeros((16,), jnp.float32)

  xbufs = (xb0, xb1, xb2)
  gbufs = (gb0, gb1, gb2)
  hbufs = (hb0, hb1, hb2)
  sems = (sem0, sem1, sem2)

  def start(st, nrows, slot):
    pltpu.async_copy(xt_hbm.at[:, pl.ds(st, nrows)],
                     xbufs[slot].at[:, pl.ds(0, nrows)], sems[slot])
    pltpu.async_copy(g_hbm.at[pl.ds(st, nrows)],
                     gbufs[slot].at[pl.ds(0, nrows)], sems[slot])
    pltpu.async_copy(h_hbm.at[pl.ds(st, nrows)],
                     hbufs[slot].at[pl.ds(0, nrows)], sems[slot])

  def wait_slot(slot, nrows):
    pltpu.make_async_copy(xt_hbm.at[:, pl.ds(0, nrows)],
                          xbufs[slot].at[:, pl.ds(0, nrows)],
                          sems[slot]).wait()
    pltpu.make_async_copy(g_hbm.at[pl.ds(0, nrows)],
                          gbufs[slot].at[pl.ds(0, nrows)], sems[slot]).wait()
    pltpu.make_async_copy(h_hbm.at[pl.ds(0, nrows)],
                          hbufs[slot].at[pl.ds(0, nrows)], sems[slot]).wait()

  def process(slot, nrows):
    xb = xbufs[slot]
    gb = gbufs[slot]
    hb = hbufs[slot]

    @plsc.parallel_loop(0, nrows // 16, 1, unroll=2)
    def _grp(gi):
      r = gi * 16
      gv = gb[pl.ds(r, 16)]
      hv = hb[pl.ds(r, 16)]
      for f in range(0, F, 2):
        xv0 = xb[f, pl.ds(r, 16)]
        xv1 = xb[f + 1, pl.ds(r, 16)]
        idx0 = xv0 + (f * NBIN)
        idx1 = xv1 + ((f + 1) * NBIN)
        plsc.addupdate_scatter(histg, [idx0], gv)
        plsc.addupdate_scatter(histg, [idx1], gv)
        plsc.addupdate_scatter(histh, [idx0], hv)
        plsc.addupdate_scatter(histh, [idx1], hv)

  start(base, CH, 0)
  start(base + CH, CH, 1)

  @plsc.parallel_loop(0, HIST // 16, 1, unroll=8)
  def _zero(i):
    histg[pl.ds(i * 16, 16)] = zeros16
    histh[pl.ds(i * 16, 16)] = zeros16

  def outer(j, carry):
    for k in range(NBUF):
      ci = j * NBUF + k
      wait_slot(k, CH)

      @pl.when(ci + 2 < NCHUNK)
      def _():
        start(base + (ci + 2) * CH, CH, (k + 2) % NBUF)

      @pl.when(ci + 2 == NCHUNK)
      def _():
        start(base + NCHUNK * CH, HALF_CH, (k + 2) % NBUF)

      process(k, CH)
    return carry
  lax.fori_loop(0, NOUTER, outer, 0)

  wait_slot(0, HALF_CH)
  process(0, HALF_CH)

  @pl.when(wid == 0)
  def _tail():
    start(TAIL_BASE, TAIL1, 1)
    wait_slot(1, TAIL1)
    process(1, TAIL1)

  pltpu.sync_copy(histg, pg_hbm.at[pl.ds(wid * HIST, HIST)])
  pltpu.sync_copy(histh, ph_hbm.at[pl.ds(wid * HIST, HIST)])


def _tc_body(pg_ref, ph_ref, xt_ref, gt_ref, ht_ref, gl_ref, hl_ref):
  xt = jnp.broadcast_to(xt_ref[...][:, :, None], (TAIL2, F, NBIN))
  bins = lax.broadcasted_iota(jnp.int32, (TAIL2, F, NBIN), 2)
  m = (xt == bins).astype(jnp.float32)
  hg_t = jnp.sum(m * gt_ref[...][:, None, None], axis=0)
  hh_t = jnp.sum(m * ht_ref[...][:, None, None], axis=0)
  hg = jnp.sum(pg_ref[...], axis=0) + hg_t
  hh = jnp.sum(ph_ref[...], axis=0) + hh_t
  rows = lax.broadcasted_iota(jnp.int32, (NBIN, NBIN), 0)
  cols = lax.broadcasted_iota(jnp.int32, (NBIN, NBIN), 1)
  tri = (rows <= cols).astype(jnp.float32)
  gl_ref[...] = jnp.dot(hg, tri, preferred_element_type=jnp.float32)
  hl_ref[...] = jnp.dot(hh, tri, preferred_element_type=jnp.float32)


@jax.jit
def kernel(X, gradient, hessian):
  mesh = plsc.VectorSubcoreMesh(core_axis_name="c", subcore_axis_name="s")
  sc = pl.kernel(
      _sc_body,
      out_type=(
          jax.ShapeDtypeStruct((NTILES * HIST,), jnp.float32),
          jax.ShapeDtypeStruct((NTILES * HIST,), jnp.float32),
      ),
      mesh=mesh,
      compiler_params=pltpu.CompilerParams(needs_layout_passes=False),
      scratch_types=[
          pltpu.VMEM((F, XPAD), jnp.int32),
          pltpu.VMEM((F, XPAD), jnp.int32),
          pltpu.VMEM((F, XPAD), jnp.int32),
          pltpu.VMEM((GBUF,), jnp.float32),
          pltpu.VMEM((GBUF,), jnp.float32),
          pltpu.VMEM((GBUF,), jnp.float32),
          pltpu.VMEM((GBUF,), jnp.float32),
          pltpu.VMEM((GBUF,), jnp.float32),
          pltpu.VMEM((GBUF,), jnp.float32),
          pltpu.VMEM((HIST,), jnp.float32),
          pltpu.VMEM((HIST,), jnp.float32),
          pltpu.SemaphoreType.DMA,
          pltpu.SemaphoreType.DMA,
          pltpu.SemaphoreType.DMA,
      ],
  )
  pg, ph = sc(X.T, gradient, hessian)

  pg3 = pg.reshape(NTILES, F, NBIN)
  ph3 = ph.reshape(NTILES, F, NBIN)
  gl, hl = pl.pallas_call(
      _tc_body,
      out_shape=(
          jax.ShapeDtypeStruct((F, NBIN), jnp.float32),
          jax.ShapeDtypeStruct((F, NBIN), jnp.float32),
      ),
  )(pg3, ph3, X[TAIL_BASE + TAIL1:], gradient[TAIL_BASE + TAIL1:],
    hessian[TAIL_BASE + TAIL1:])
  return (gl[None], hl[None])
